# baseline (speedup 1.0000x reference)
import jax
import jax.numpy as jnp
from jax.experimental import pallas as pl


def _lin(x, p):
    return x @ p["W"] + p["b"]


def _layer(p, h, x, src, dst, ea, fixed_mask):
    diff = x[src] - x[dst]
    dist_sq = jnp.sum(diff * diff, axis=-1, keepdims=True) / 100.0
    m_in = jnp.concatenate([h[src], h[dst], dist_sq, ea], axis=-1)
    m = jax.nn.silu(_lin(jax.nn.silu(_lin(m_in, p["phi_e1"])), p["phi_e2"]))
    cw = jnp.tanh(_lin(jax.nn.silu(_lin(m, p["phi_x1"])), p["phi_x2"]))
    cd = diff * cw
    coord_agg = jnp.zeros_like(x).at[dst].add(cd)
    coord_agg = coord_agg * (~fixed_mask).astype(x.dtype)[:, None]
    msg_agg = jnp.zeros((h.shape[0], m.shape[-1]), dtype=h.dtype).at[dst].add(m)
    h_new = h + _lin(jax.nn.silu(_lin(jnp.concatenate([h, msg_agg], axis=-1), p["phi_h1"])), p["phi_h2"])
    x_new = x + coord_agg
    return h_new, x_new


def kernel(lig_x, lig_h, poc_x, poc_h, lig_edge_index, lig_edge_attr, poc_edge_index, poc_edge_attr, cross_edge_index, cross_edge_attr, t, params):
    N = lig_h.shape[0]
    M = poc_h.shape[0]
    t_feat = jnp.broadcast_to(t.reshape(1, 1), (N, 1))
    h_lig = _lin(jnp.concatenate([lig_h, t_feat], axis=-1), params["lig_emb"])
    h_poc = _lin(poc_h, params["poc_emb"])
    h = jnp.concatenate([h_lig, h_poc], axis=0)
    x = jnp.concatenate([lig_x, poc_x], axis=0)
    x0 = lig_x
    poc_sh = poc_edge_index + N
    cross_sh = jnp.stack([cross_edge_index[0] + N, cross_edge_index[1]], axis=0)
    ei = jnp.concatenate([lig_edge_index, poc_sh, cross_sh], axis=1)
    ea = jnp.concatenate([
        _lin(lig_edge_attr, params["lig_edge_emb"]),
        _lin(poc_edge_attr, params["poc_edge_emb"]),
        _lin(cross_edge_attr, params["cross_edge_emb"]),
    ], axis=0)
    fixed_mask = jnp.concatenate([jnp.zeros(N, dtype=bool), jnp.ones(M, dtype=bool)], axis=0)
    src, dst = ei[0], ei[1]
    for lp in params["layers"]:
        h, x = _layer(lp, h, x, src, dst, ea, fixed_mask)
    disp = x[:N] - x0
    scale = _lin(jax.nn.silu(_lin(h[:N], params["out1"])), params["out2"])
    return scale * disp


# speedup vs baseline: 1.2309x; 1.2309x over previous
import functools

import jax
import jax.numpy as jnp
from jax import lax
from jax.experimental import pallas as pl
from jax.experimental.pallas import tpu as pltpu
from jax.experimental.pallas import tpu_sc as plsc

F32 = jnp.float32
N = 25000
V = 50000
E = 800000
HID = 64
TW = 128
PW = 128
NB = 1000
EB = 2000
CH = 128
NCHUNK = E // CH
NW = 32
RNG = 13000
DRW = 13120
SCH = 64
SNCHUNK = E // SCH
DC = 32
NZC = 26
RZ = 16 * NZC * DC


def _dot(a, b):
    return jnp.dot(a, b, preferred_element_type=F32,
                   precision=lax.Precision.HIGHEST)


def _prep_body(hin, wemb, bemb, xc, wsrc, wdst, b1, h0, ts, td):
    h = _dot(hin[...], wemb[0]) + bemb[0]
    h0[...] = h
    x = xc[...]
    z = jnp.zeros((NB, TW - HID - 3), F32)
    hs = _dot(h, wsrc[...])
    hd = _dot(h, wdst[...]) + b1[...]
    ts[...] = jnp.concatenate([hs, x, z], axis=1)
    td[...] = jnp.concatenate([hd, -x, z], axis=1)


def _prep_call(hin, wemb, bemb, xc, wsrc, wdst, b1):
    n_blk = V // NB
    return pl.pallas_call(
        _prep_body,
        grid=(n_blk,),
        in_specs=[
            pl.BlockSpec((NB, 32), lambda i: (i, 0)),
            pl.BlockSpec((1, 32, HID),
                         lambda i: ((i >= N // NB).astype(jnp.int32), 0, 0)),
            pl.BlockSpec((1, 1, HID),
                         lambda i: ((i >= N // NB).astype(jnp.int32), 0, 0)),
            pl.BlockSpec((NB, 3), lambda i: (i, 0)),
            pl.BlockSpec((HID, HID), lambda i: (0, 0)),
            pl.BlockSpec((HID, HID), lambda i: (0, 0)),
            pl.BlockSpec((1, HID), lambda i: (0, 0)),
        ],
        out_specs=(
            pl.BlockSpec((NB, HID), lambda i: (i, 0)),
            pl.BlockSpec((NB, TW), lambda i: (i, 0)),
            pl.BlockSpec((NB, TW), lambda i: (i, 0)),
        ),
        out_shape=(
            jax.ShapeDtypeStruct((V, HID), F32),
            jax.ShapeDtypeStruct((V, TW), F32),
            jax.ShapeDtypeStruct((V, TW), F32),
        ),
    )(hin, wemb, bemb, xc, wsrc, wdst, b1)


def _edge_body(g, ein, ew, cv, w1d, w2, b2, wx1, bx1, wx2, bx2, pm):
    gv = g[...]
    g64 = gv[:, :HID]
    diff = gv[:, HID:HID + 3]
    dsq = jnp.sum(diff * diff, axis=1, keepdims=True) * 0.01
    pre = g64 + _dot(ein[...], ew[0]) + cv[0] + dsq * w1d[...]
    t1 = jax.nn.silu(pre)
    m = jax.nn.silu(_dot(t1, w2[...]) + b2[...])
    vv = jax.nn.silu(_dot(m, wx1[...]) + bx1[...])
    cw = jnp.tanh(_dot(vv, wx2[...]) + bx2[...])
    cd = diff * cw
    pm[...] = jnp.concatenate(
        [m, cd, jnp.zeros((EB, PW - HID - 3), F32)], axis=1)


def _edge_call(g, ein, ew, cv, w1d, w2, b2, wx1, bx1, wx2, bx2):
    n_blk = E // EB

    def _seg(i):
        return ((i >= 400000 // EB).astype(jnp.int32)
                + (i >= 650000 // EB).astype(jnp.int32))

    return pl.pallas_call(
        _edge_body,
        grid=(n_blk,),
        in_specs=[
            pl.BlockSpec((EB, TW), lambda i: (i, 0)),
            pl.BlockSpec((EB, 16), lambda i: (i, 0)),
            pl.BlockSpec((1, 16, HID), lambda i: (_seg(i), 0, 0)),
            pl.BlockSpec((1, 1, HID), lambda i: (_seg(i), 0, 0)),
            pl.BlockSpec((1, HID), lambda i: (0, 0)),
            pl.BlockSpec((HID, HID), lambda i: (0, 0)),
            pl.BlockSpec((1, HID), lambda i: (0, 0)),
            pl.BlockSpec((HID, HID), lambda i: (0, 0)),
            pl.BlockSpec((1, HID), lambda i: (0, 0)),
            pl.BlockSpec((HID, 1), lambda i: (0, 0)),
            pl.BlockSpec((1, 1), lambda i: (0, 0)),
        ],
        out_specs=pl.BlockSpec((EB, PW), lambda i: (i, 0)),
        out_shape=jax.ShapeDtypeStruct((E, PW), F32),
    )(g, ein, ew, cv, w1d, w2, b2, wx1, bx1, wx2, bx2)


def _aggk_idx(k):
    def _f(i):
        return (jnp.clip(i - (RNG // NB) * k, 0, RNG // NB - 1), 0)
    return _f


def _agg_select(rid, a0, a1, a2, a3):
    return jnp.where(rid < RNG, a0,
                     jnp.where(rid < 2 * RNG, a1,
                               jnp.where(rid < 3 * RNG, a2, a3)))


def _node_body(h, agg0, agg1, agg2, agg3, tsp, whh, whm, bh1, wh2, bh2,
               wsrc, wdst, b1, h2o, tso, tdo):
    i = pl.program_id(0)
    hv = h[...]
    rid = i * NB + lax.broadcasted_iota(jnp.int32, (NB, 1), 0)
    maskf = (rid < N).astype(F32)
    av = _agg_select(rid, agg0[...], agg1[...], agg2[...], agg3[...])
    msg = av[:, :HID]
    coord = av[:, HID:HID + 3]
    u = jax.nn.silu(_dot(hv, whh[...]) + _dot(msg, whm[...]) + bh1[...])
    h2 = hv + _dot(u, wh2[...]) + bh2[...]
    x = tsp[...][:, HID:HID + 3] + coord * maskf
    z = jnp.zeros((NB, TW - HID - 3), F32)
    hs = _dot(h2, wsrc[...])
    hd = _dot(h2, wdst[...]) + b1[...]
    h2o[...] = h2
    tso[...] = jnp.concatenate([hs, x, z], axis=1)
    tdo[...] = jnp.concatenate([hd, -x, z], axis=1)


def _node_call(h, agg0, agg1, agg2, agg3, tsp, whh, whm, bh1, wh2, bh2,
               wsrc, wdst, b1):
    n_blk = V // NB
    return pl.pallas_call(
        _node_body,
        grid=(n_blk,),
        in_specs=[
            pl.BlockSpec((NB, HID), lambda i: (i, 0)),
            pl.BlockSpec((NB, PW), _aggk_idx(0)),
            pl.BlockSpec((NB, PW), _aggk_idx(1)),
            pl.BlockSpec((NB, PW), _aggk_idx(2)),
            pl.BlockSpec((NB, PW), _aggk_idx(3)),
            pl.BlockSpec((NB, TW), lambda i: (i, 0)),
            pl.BlockSpec((HID, HID), lambda i: (0, 0)),
            pl.BlockSpec((HID, HID), lambda i: (0, 0)),
            pl.BlockSpec((1, HID), lambda i: (0, 0)),
            pl.BlockSpec((HID, HID), lambda i: (0, 0)),
            pl.BlockSpec((1, HID), lambda i: (0, 0)),
            pl.BlockSpec((HID, HID), lambda i: (0, 0)),
            pl.BlockSpec((HID, HID), lambda i: (0, 0)),
            pl.BlockSpec((1, HID), lambda i: (0, 0)),
        ],
        out_specs=(
            pl.BlockSpec((NB, HID), lambda i: (i, 0)),
            pl.BlockSpec((NB, TW), lambda i: (i, 0)),
            pl.BlockSpec((NB, TW), lambda i: (i, 0)),
        ),
        out_shape=(
            jax.ShapeDtypeStruct((V, HID), F32),
            jax.ShapeDtypeStruct((V, TW), F32),
            jax.ShapeDtypeStruct((V, TW), F32),
        ),
    )(h, agg0, agg1, agg2, agg3, tsp, whh, whm, bh1, wh2, bh2,
      wsrc, wdst, b1)


def _head_body(h, agg0, agg1, agg2, agg3, tsp, x0, whh, whm, bh1, wh2,
               bh2, wo1, bo1, wo2, bo2, out):
    i = pl.program_id(0)
    hv = h[...]
    rid = i * NB + lax.broadcasted_iota(jnp.int32, (NB, 1), 0)
    maskf = (rid < N).astype(F32)
    av = _agg_select(rid, agg0[...], agg1[...], agg2[...], agg3[...])
    msg = av[:, :HID]
    coord = av[:, HID:HID + 3]
    u = jax.nn.silu(_dot(hv, whh[...]) + _dot(msg, whm[...]) + bh1[...])
    h2 = hv + _dot(u, wh2[...]) + bh2[...]
    x = tsp[...][:, HID:HID + 3] + coord * maskf
    disp = x - x0[...]
    s = jax.nn.silu(_dot(h2, wo1[...]) + bo1[...])
    scale = _dot(s, wo2[...]) + bo2[...]
    out[...] = scale * disp * maskf


def _head_call(h, agg0, agg1, agg2, agg3, tsp, x0, whh, whm, bh1, wh2,
               bh2, wo1, bo1, wo2, bo2):
    n_blk = V // NB
    return pl.pallas_call(
        _head_body,
        grid=(n_blk,),
        in_specs=[
            pl.BlockSpec((NB, HID), lambda i: (i, 0)),
            pl.BlockSpec((NB, PW), _aggk_idx(0)),
            pl.BlockSpec((NB, PW), _aggk_idx(1)),
            pl.BlockSpec((NB, PW), _aggk_idx(2)),
            pl.BlockSpec((NB, PW), _aggk_idx(3)),
            pl.BlockSpec((NB, TW), lambda i: (i, 0)),
            pl.BlockSpec((NB, 3), lambda i: (i, 0)),
            pl.BlockSpec((HID, HID), lambda i: (0, 0)),
            pl.BlockSpec((HID, HID), lambda i: (0, 0)),
            pl.BlockSpec((1, HID), lambda i: (0, 0)),
            pl.BlockSpec((HID, HID), lambda i: (0, 0)),
            pl.BlockSpec((1, HID), lambda i: (0, 0)),
            pl.BlockSpec((HID, HID), lambda i: (0, 0)),
            pl.BlockSpec((1, HID), lambda i: (0, 0)),
            pl.BlockSpec((HID, 1), lambda i: (0, 0)),
            pl.BlockSpec((1, 1), lambda i: (0, 0)),
        ],
        out_specs=pl.BlockSpec((NB, 3), lambda i: (i, 0)),
        out_shape=jax.ShapeDtypeStruct((V, 3), F32),
    )(h, agg0, agg1, agg2, agg3, tsp, x0, whh, whm, bh1, wh2, bh2,
      wo1, bo1, wo2, bo2)


@functools.lru_cache(maxsize=1)
def _mesh():
    return plsc.VectorSubcoreMesh(core_axis_name="c", subcore_axis_name="s")


def _gather_body(ts, td, srcr, dstr, g, idxs, idxd, bufs, bufd, sem1, sem2):
    c = lax.axis_index("c")
    s = lax.axis_index("s")
    wid = s * 2 + c
    n_lo = NCHUNK // NW
    n_extra = NCHUNK - n_lo * NW
    nch = jnp.where(wid < n_extra, n_lo + 1, n_lo)

    @pl.loop(0, nch)
    def _chunk(k):
        j = wid + NW * k
        off = j * CH
        pltpu.sync_copy(srcr.at[pl.ds(off, CH)], idxs)
        pltpu.sync_copy(dstr.at[pl.ds(off, CH)], idxd)
        cp1 = pltpu.async_copy(ts.at[idxs], bufs, sem1)
        cp2 = pltpu.async_copy(td.at[idxd], bufd, sem2)
        cp1.wait()
        cp2.wait()

        @pl.loop(0, CH, unroll=4)
        def _row(r):
            for kk in range(TW // 16):
                sl = pl.ds(kk * 16, 16)
                bufs[r, sl] = bufs[r, sl] + bufd[r, sl]

        pltpu.sync_copy(bufs, g.at[pl.ds(off, CH)])


def _gather_call(ts, td, src, dst):
    fn = pl.kernel(
        _gather_body,
        out_type=jax.ShapeDtypeStruct((E, TW), F32),
        mesh=_mesh(),
        scratch_types=[
            pltpu.VMEM((CH,), jnp.int32),
            pltpu.VMEM((CH,), jnp.int32),
            pltpu.VMEM((CH, TW), F32),
            pltpu.VMEM((CH, TW), F32),
            pltpu.SemaphoreType.DMA,
            pltpu.SemaphoreType.DMA,
        ],
    )
    return fn(ts, td, src, dst)


def _scatter_body_generic(p, l0r, l1r, zidx_h, agg0, agg1, idxv, zi, pbuf,
                          dbuf, acc, sem):
    c = lax.axis_index("c")
    s = lax.axis_index("s")
    zeros16 = jnp.zeros((16,), F32)

    @pl.loop(0, DC, unroll=4)
    def _z(r):
        for col in range(0, PW, 16):
            dbuf[r, pl.ds(col, 16)] = zeros16

    for t in range(NZC):
        ci = s * NZC + t
        pltpu.sync_copy(zidx_h.at[pl.ds(ci * DC, DC)], zi)
        pltpu.sync_copy(dbuf, acc.at[zi])

    plsc.subcore_barrier()

    n_lo = SNCHUNK // 16
    n_extra = SNCHUNK - n_lo * 16
    nch = jnp.where(s < n_extra, n_lo + 1, n_lo)

    @pl.loop(0, nch)
    def _chunk(k):
        j = s + 16 * k
        off = j * SCH

        @pl.when(c == 0)
        def _l0():
            pltpu.sync_copy(l0r.at[pl.ds(off, SCH)], idxv)

        @pl.when(c == 1)
        def _l1():
            pltpu.sync_copy(l1r.at[pl.ds(off, SCH)], idxv)

        pltpu.sync_copy(p.at[pl.ds(off, SCH)], pbuf)
        pltpu.sync_copy(pbuf, acc.at[idxv], add=True)

    plsc.subcore_barrier()

    for t in range(NZC):
        ci = s * NZC + t
        pltpu.sync_copy(zidx_h.at[pl.ds(ci * DC, DC)], zi)
        pltpu.sync_copy(acc.at[zi], dbuf)

        @pl.when(c == 0)
        def _d0():
            pltpu.sync_copy(dbuf, agg0.at[pl.ds(ci * DC, DC)])

        @pl.when(c == 1)
        def _d1():
            pltpu.sync_copy(dbuf, agg1.at[pl.ds(ci * DC, DC)])


def _scatter_call(p, lidx_a, lidx_b, zidx_h):
    fn = pl.kernel(
        _scatter_body_generic,
        out_type=(jax.ShapeDtypeStruct((RZ, PW), F32),
                  jax.ShapeDtypeStruct((RZ, PW), F32)),
        mesh=_mesh(),
        scratch_types=[
            pltpu.VMEM((SCH,), jnp.int32),
            pltpu.VMEM((DC,), jnp.int32),
            pltpu.VMEM((SCH, PW), F32),
            pltpu.VMEM((DC, PW), F32),
            pltpu.VMEM_SHARED((DRW, PW), F32),
            pltpu.SemaphoreType.DMA,
        ],
    )
    return fn(p, lidx_a, lidx_b, zidx_h)


def _pad_cols(a, w):
    return jnp.pad(a, ((0, 0), (0, w - a.shape[1])))


def _pad_rows(a, r):
    return jnp.pad(a, ((0, r - a.shape[0]), (0, 0)))


def kernel(lig_x, lig_h, poc_x, poc_h, lig_edge_index, lig_edge_attr,
           poc_edge_index, poc_edge_attr, cross_edge_index, cross_edge_attr,
           t, params):
    p = params

    x_cat = jnp.concatenate([lig_x, poc_x], axis=0)
    t_feat = jnp.broadcast_to(t.reshape(1, 1), (N, 1))
    hin = jnp.concatenate([
        _pad_cols(jnp.concatenate([lig_h, t_feat], axis=1), 32),
        _pad_cols(poc_h, 32),
    ], axis=0)
    wemb = jnp.stack([
        _pad_rows(p["lig_emb"]["W"], 32),
        _pad_rows(p["poc_emb"]["W"], 32),
    ])
    bemb = jnp.stack([p["lig_emb"]["b"], p["poc_emb"]["b"]]).reshape(2, 1, HID)

    ein = jnp.concatenate([
        _pad_cols(lig_edge_attr, 16),
        poc_edge_attr,
        cross_edge_attr,
    ], axis=0)

    src = jnp.concatenate([
        lig_edge_index[0], poc_edge_index[0] + N, cross_edge_index[0] + N])
    dst = jnp.concatenate([
        lig_edge_index[1], poc_edge_index[1] + N, cross_edge_index[1]])
    src = src.astype(jnp.int32)
    dst = dst.astype(jnp.int32)
    dummyv = RNG + 8 + (jnp.arange(E, dtype=jnp.int32) % 64)
    lidx = [jnp.where((dst >= k * RNG) & (dst < (k + 1) * RNG),
                      dst - k * RNG, dummyv) for k in range(4)]
    zidx = jnp.arange(RZ, dtype=jnp.int32) % DRW

    e_ws = [_pad_rows(p["lig_edge_emb"]["W"], 16),
            p["poc_edge_emb"]["W"],
            p["cross_edge_emb"]["W"]]
    e_bs = [p["lig_edge_emb"]["b"], p["poc_edge_emb"]["b"],
            p["cross_edge_emb"]["b"]]

    lw = []
    for lp in p["layers"]:
        w1 = lp["phi_e1"]["W"]
        d = {
            "wsrc": w1[:HID],
            "wdst": w1[HID:2 * HID],
            "w1d": w1[2 * HID:2 * HID + 1],
            "b1": lp["phi_e1"]["b"].reshape(1, HID),
            "ew": jnp.stack([_dot(we, w1[2 * HID + 1:]) for we in e_ws]),
            "cv": jnp.stack([_dot(be[None, :], w1[2 * HID + 1:])
                             for be in e_bs]),
            "w2": lp["phi_e2"]["W"],
            "b2": lp["phi_e2"]["b"].reshape(1, HID),
            "wx1": lp["phi_x1"]["W"],
            "bx1": lp["phi_x1"]["b"].reshape(1, HID),
            "wx2": lp["phi_x2"]["W"],
            "bx2": lp["phi_x2"]["b"].reshape(1, 1),
            "whh": lp["phi_h1"]["W"][:HID],
            "whm": lp["phi_h1"]["W"][HID:],
            "bh1": lp["phi_h1"]["b"].reshape(1, HID),
            "wh2": lp["phi_h2"]["W"],
            "bh2": lp["phi_h2"]["b"].reshape(1, HID),
        }
        lw.append(d)

    h, ts_t, td_t = _prep_call(hin, wemb, bemb, x_cat,
                               lw[0]["wsrc"], lw[0]["wdst"], lw[0]["b1"])
    out = None
    for l in range(4):
        d = lw[l]
        g = _gather_call(ts_t, td_t, src, dst)
        pay = _edge_call(g, ein, d["ew"], d["cv"], d["w1d"], d["w2"],
                         d["b2"], d["wx1"], d["bx1"], d["wx2"], d["bx2"])
        agg0, agg1 = _scatter_call(pay, lidx[0], lidx[1], zidx)
        agg2, agg3 = _scatter_call(pay, lidx[2], lidx[3], zidx)
        if l < 3:
            nd = lw[l + 1]
            h, ts_t, td_t = _node_call(h, agg0, agg1, agg2, agg3, ts_t,
                                       d["whh"], d["whm"], d["bh1"], d["wh2"],
                                       d["bh2"], nd["wsrc"], nd["wdst"],
                                       nd["b1"])
        else:
            out = _head_call(h, agg0, agg1, agg2, agg3, ts_t, x_cat,
                             d["whh"], d["whm"], d["bh1"], d["wh2"], d["bh2"],
                             p["out1"]["W"], p["out1"]["b"].reshape(1, HID),
                             p["out2"]["W"], p["out2"]["b"].reshape(1, 1))
    return out[:N]
